# baseline (speedup 1.0000x reference)
import functools

import jax
import jax.numpy as jnp
from jax import lax
from jax.experimental import pallas as pl
from jax.experimental.pallas import tpu as pltpu
from jax.experimental.pallas import tpu_sc as plsc

_B, _S, _ML, _D, _F = 16, 512, 2048, 256, 256
_NROWS = _B * _ML
_ZROW = _B * _S
_NW = 32
_RPW = _NROWS // _NW
_CH = 32
_NCH = _RPW // _CH
_NBUF = 4


def _conv3(xin, wc_ref, b_ref):
    z = jnp.dot(xin, wc_ref[:, :], preferred_element_type=jnp.float32)
    zero = jnp.zeros((1, _F), jnp.float32)
    y = z[:, _F:2 * _F]
    y = y + jnp.concatenate([zero, z[:-1, 0:_F]], axis=0)
    y = y + jnp.concatenate([z[1:, 2 * _F:], zero], axis=0)
    return y + b_ref[:, :]


def _ln(h, g_ref, be_ref):
    m = jnp.mean(h, axis=-1, keepdims=True)
    d = h - m
    v = jnp.mean(d * d, axis=-1, keepdims=True)
    return d * jax.lax.rsqrt(v + 1e-5) * g_ref[:, :] + be_ref[:, :]


def _predictor(xin, w1, b1, g1, be1, w2, b2, g2, be2, lwt, lb):
    h = jnp.maximum(_conv3(xin, w1, b1), 0.0)
    h = _ln(h, g1, be1)
    h = jnp.maximum(_conv3(h, w2, b2), 0.0)
    h = _ln(h, g2, be2)
    out = jnp.sum(h * lwt[:, :], axis=-1, keepdims=True)
    return out + lb[:, :]


def _pack_params(p):
    w1c = jnp.concatenate([p['w1'][0], p['w1'][1], p['w1'][2]], axis=1)
    w2c = jnp.concatenate([p['w2'][0], p['w2'][1], p['w2'][2]], axis=1)
    return [
        w1c, p['b1'].reshape(1, _F), p['g1'].reshape(1, _F), p['be1'].reshape(1, _F),
        w2c, p['b2'].reshape(1, _F), p['g2'].reshape(1, _F), p['be2'].reshape(1, _F),
        p['lw'].reshape(1, _F), p['lb'].reshape(1, 1),
    ]


def _full_spec(arr):
    return pl.BlockSpec(arr.shape, lambda *_: (0,) * arr.ndim)


def _cum_and_gmat(dur_row):
    jj = lax.broadcasted_iota(jnp.int32, (_S, _S), 0)
    ss = lax.broadcasted_iota(jnp.int32, (_S, _S), 1)
    tri = (jj <= ss).astype(jnp.float32)
    cum_row = jnp.dot(dur_row, tri, preferred_element_type=jnp.float32)
    cumprev_row = cum_row - dur_row
    t_col = lax.broadcasted_iota(jnp.int32, (_ML, 1), 0).astype(jnp.float32)
    gmat = (t_col < cum_row).astype(jnp.float32) - (t_col < cumprev_row).astype(jnp.float32)
    return cum_row, gmat


def _idx_body(dur_ref, pt_ref, et_ref, pbins_ref, ebins_ref,
              mel_ref, idx_ref):
    b = pl.program_id(0)
    dur_row = dur_ref[0].astype(jnp.float32)
    jj = lax.broadcasted_iota(jnp.int32, (_S, _S), 0)
    ss = lax.broadcasted_iota(jnp.int32, (_S, _S), 1)
    tri = (jj <= ss).astype(jnp.float32)
    cum_row = jnp.dot(dur_row, tri, preferred_element_type=jnp.float32)
    t_col = lax.broadcasted_iota(jnp.int32, (_ML, 1), 0).astype(jnp.float32)

    src = jnp.sum((cum_row <= t_col).astype(jnp.float32), axis=-1, keepdims=True)
    mel_f = cum_row[:, _S - 1:]
    valid = t_col < jnp.minimum(mel_f, jnp.float32(_ML))
    base_f = (b * _S).astype(jnp.float32)
    ti = lax.broadcasted_iota(jnp.int32, (_ML, 1), 0)
    zrow = _ZROW + jnp.bitwise_and(ti, 63)
    idx_ref[0, 0] = jnp.where(valid, (src + base_f).astype(jnp.int32), zrow)

    pidx = jnp.sum((pbins_ref[:, :] < pt_ref[0]).astype(jnp.float32),
                   axis=-1, keepdims=True)
    eidx = jnp.sum((ebins_ref[:, :] < et_ref[0]).astype(jnp.float32),
                   axis=-1, keepdims=True)
    idx_ref[1, 0] = pidx.astype(jnp.int32)
    idx_ref[2, 0] = eidx.astype(jnp.int32)
    mel_ref[0] = cum_row[:, _S - 128:]


def _sc_xout_body(xpad_hbm, pemb_hbm, eemb_hbm, idx_hbm,
                  xout_hbm, gix, pix, eix, *bufs_and_sems):
    xbufs = bufs_and_sems[0:_NBUF]
    pbufs = bufs_and_sems[_NBUF:2 * _NBUF]
    ebufs = bufs_and_sems[2 * _NBUF:3 * _NBUF]
    gsems = bufs_and_sems[3 * _NBUF:4 * _NBUF]
    wsems = bufs_and_sems[4 * _NBUF:5 * _NBUF]
    wid = lax.axis_index("s") * 2 + lax.axis_index("c")
    pltpu.sync_copy(idx_hbm.at[0, wid], gix)
    pltpu.sync_copy(idx_hbm.at[1, wid], pix)
    pltpu.sync_copy(idx_hbm.at[2, wid], eix)

    def fire(k):
        m = k % _NBUF
        pltpu.async_copy(xpad_hbm.at[gix.at[k]], xbufs[m], gsems[m])
        pltpu.async_copy(pemb_hbm.at[pix.at[k]], pbufs[m], gsems[m])
        pltpu.async_copy(eemb_hbm.at[eix.at[k]], ebufs[m], gsems[m])

    def wait_gather(k):
        m = k % _NBUF
        pltpu.make_async_copy(xpad_hbm.at[gix.at[k]], xbufs[m], gsems[m]).wait()
        pltpu.make_async_copy(pemb_hbm.at[pix.at[k]], pbufs[m], gsems[m]).wait()
        pltpu.make_async_copy(eemb_hbm.at[eix.at[k]], ebufs[m], gsems[m]).wait()

    def out_at(k):
        return xout_hbm.at[pl.ds(wid * _RPW + k * _CH, _CH)]

    def wait_write(k):
        pltpu.make_async_copy(xbufs[k % _NBUF], out_at(k), wsems[k % _NBUF]).wait()

    for k in range(_NBUF - 1):
        fire(k)
    for k in range(_NCH):
        m = k % _NBUF
        if k + _NBUF - 1 < _NCH:
            if k >= 1:
                wait_write(k - 1)
            fire(k + _NBUF - 1)
        wait_gather(k)

        def row(r, c):
            for j in range(_D // 16):
                sl = pl.ds(j * 16, 16)
                xbufs[m][r, sl] = xbufs[m][r, sl] + pbufs[m][r, sl] + ebufs[m][r, sl]
            return c

        lax.fori_loop(0, _CH, row, 0)
        pltpu.async_copy(xbufs[m], out_at(k), wsems[m])
    for k in range(_NCH - _NBUF, _NCH):
        if k >= 0:
            wait_write(k)


def _main_body(xf_ref, dur_ref,
               dw1, db1, dg1, dbe1, dw2, db2, dg2, dbe2, dlwt, dlb,
               pw1, pb1, pg1, pbe1, pw2, pb2, pg2, pbe2, plwt, plb,
               ew1, eb1, eg1, ebe1, ew2, eb2, eg2, ebe2, elwt, elb,
               logdur_ref, ppred_ref, epred_ref):
    xf_b = xf_ref[0]
    dur_row = dur_ref[0].astype(jnp.float32)

    logdur_ref[0] = _predictor(xf_b, dw1, db1, dg1, dbe1, dw2, db2, dg2, dbe2, dlwt, dlb)

    _, gmat = _cum_and_gmat(dur_row)
    xf_exp = jnp.dot(gmat, xf_b, preferred_element_type=jnp.float32)

    ppred_ref[0] = _predictor(xf_exp, pw1, pb1, pg1, pbe1, pw2, pb2, pg2, pbe2, plwt, plb)
    epred_ref[0] = _predictor(xf_exp, ew1, eb1, eg1, ebe1, ew2, eb2, eg2, ebe2, elwt, elb)


def kernel(x, x_features, src_mask, mel_mask, duration_target, pitch_target,
           energy_target, max_len, dur_params, pitch_params, energy_params,
           pitch_bins, energy_bins, pitch_embedding, energy_embedding):
    B, S, D = x.shape
    ML = mel_mask.shape[1]

    dur3 = duration_target.reshape(B, 1, S)
    pt3 = pitch_target.reshape(B, ML, 1)
    et3 = energy_target.reshape(B, ML, 1)
    pad = jnp.full((1,), jnp.inf, jnp.float32)
    pbins = jnp.concatenate([pitch_bins, pad]).reshape(1, 256)
    ebins = jnp.concatenate([energy_bins, pad]).reshape(1, 256)

    batch3 = lambda i: (i, 0, 0)
    dparams = _pack_params(dur_params)
    pparams = _pack_params(pitch_params)
    eparams = _pack_params(energy_params)

    mel3, idx3 = pl.pallas_call(
        _idx_body,
        grid=(B,),
        in_specs=[
            pl.BlockSpec((1, 1, S), batch3),
            pl.BlockSpec((1, ML, 1), batch3),
            pl.BlockSpec((1, ML, 1), batch3),
            _full_spec(pbins), _full_spec(ebins),
        ],
        out_specs=[
            pl.BlockSpec((1, 1, 128), batch3),
            pl.BlockSpec((3, 1, ML, 1), lambda i: (0, i, 0, 0)),
        ],
        out_shape=[
            jax.ShapeDtypeStruct((B, 1, 128), jnp.float32),
            jax.ShapeDtypeStruct((3, B, ML, 1), jnp.int32),
        ],
    )(dur3, pt3, et3, pbins, ebins)

    xpad = jnp.pad(x.reshape(B * S, D), ((0, 64), (0, 0)))
    idx_w = idx3.reshape(3, _NW, _NCH, _CH)

    mesh = plsc.VectorSubcoreMesh(core_axis_name="c", subcore_axis_name="s")

    xout_flat = functools.partial(
        pl.kernel,
        out_type=jax.ShapeDtypeStruct((_NROWS, D), jnp.float32),
        mesh=mesh,
        scratch_types=(
            [pltpu.VMEM((_NCH, _CH), jnp.int32)] * 3
            + [pltpu.VMEM((_CH, D), jnp.float32)] * (3 * _NBUF)
            + [pltpu.SemaphoreType.DMA] * (2 * _NBUF)
        ),
    )(_sc_xout_body)(xpad, pitch_embedding, energy_embedding, idx_w)

    logdur3, ppred3, epred3 = pl.pallas_call(
        _main_body,
        grid=(B,),
        in_specs=[
            pl.BlockSpec((1, S, D), batch3),
            pl.BlockSpec((1, 1, S), batch3),
        ] + [_full_spec(a) for a in dparams + pparams + eparams],
        out_specs=[
            pl.BlockSpec((1, S, 1), batch3),
            pl.BlockSpec((1, ML, 1), batch3),
            pl.BlockSpec((1, ML, 1), batch3),
        ],
        out_shape=[
            jax.ShapeDtypeStruct((B, S, 1), jnp.float32),
            jax.ShapeDtypeStruct((B, ML, 1), jnp.float32),
            jax.ShapeDtypeStruct((B, ML, 1), jnp.float32),
        ],
    )(x_features, dur3, *dparams, *pparams, *eparams)

    x_out = xout_flat.reshape(B, ML, D)
    log_duration_prediction = logdur3.reshape(B, S)
    pitch_prediction = ppred3.reshape(B, ML)
    energy_prediction = epred3.reshape(B, ML)
    mel_len = mel3[:, 0, 127].astype(jnp.int32)

    return (x_out, log_duration_prediction, duration_target, pitch_prediction,
            energy_prediction, mel_len, mel_mask)


# speedup vs baseline: 1.0522x; 1.0522x over previous
import functools

import jax
import jax.numpy as jnp
from jax import lax
from jax.experimental import pallas as pl
from jax.experimental.pallas import tpu as pltpu
from jax.experimental.pallas import tpu_sc as plsc

_B, _S, _ML, _D, _F = 16, 512, 2048, 256, 256
_NROWS = _B * _ML
_ZROW = _B * _S
_NW = 32
_RPW = _NROWS // _NW
_CH = 32
_NCH = _RPW // _CH
_NBUF = 4


def _conv3(xin, w_ref, b_ref):
    zero = jnp.zeros((1, xin.shape[1]), jnp.float32)
    xprev = jnp.concatenate([zero, xin[:-1]], axis=0)
    xnext = jnp.concatenate([xin[1:], zero], axis=0)
    y = jnp.dot(xprev, w_ref[0], preferred_element_type=jnp.float32)
    y = y + jnp.dot(xin, w_ref[1], preferred_element_type=jnp.float32)
    y = y + jnp.dot(xnext, w_ref[2], preferred_element_type=jnp.float32)
    return y + b_ref[:, :]


def _ln(h, g_ref, be_ref):
    m = jnp.mean(h, axis=-1, keepdims=True)
    d = h - m
    v = jnp.mean(d * d, axis=-1, keepdims=True)
    return d * jax.lax.rsqrt(v + 1e-5) * g_ref[:, :] + be_ref[:, :]


def _predictor(xin, w1, b1, g1, be1, w2, b2, g2, be2, lwt, lb):
    h = jnp.maximum(_conv3(xin, w1, b1), 0.0)
    h = _ln(h, g1, be1)
    h = jnp.maximum(_conv3(h, w2, b2), 0.0)
    h = _ln(h, g2, be2)
    out = jnp.sum(h * lwt[:, :], axis=-1, keepdims=True)
    return out + lb[:, :]


def _pack_params(p):
    return [
        p['w1'], p['b1'].reshape(1, _F), p['g1'].reshape(1, _F), p['be1'].reshape(1, _F),
        p['w2'], p['b2'].reshape(1, _F), p['g2'].reshape(1, _F), p['be2'].reshape(1, _F),
        p['lw'].reshape(1, _F), p['lb'].reshape(1, 1),
    ]


def _full_spec(arr):
    return pl.BlockSpec(arr.shape, lambda *_: (0,) * arr.ndim)


def _cum_and_gmat(dur_row):
    jj = lax.broadcasted_iota(jnp.int32, (_S, _S), 0)
    ss = lax.broadcasted_iota(jnp.int32, (_S, _S), 1)
    tri = (jj <= ss).astype(jnp.float32)
    cum_row = jnp.dot(dur_row, tri, preferred_element_type=jnp.float32)
    cumprev_row = cum_row - dur_row
    t_col = lax.broadcasted_iota(jnp.int32, (_ML, 1), 0).astype(jnp.float32)
    gmat = (t_col < cum_row).astype(jnp.float32) - (t_col < cumprev_row).astype(jnp.float32)
    return cum_row, gmat


def _idx_body(dur_ref, pt_ref, et_ref, pbins_ref, ebins_ref,
              mel_ref, idx_ref):
    b = pl.program_id(0)
    dur_row = dur_ref[0].astype(jnp.float32)
    jj = lax.broadcasted_iota(jnp.int32, (_S, _S), 0)
    ss = lax.broadcasted_iota(jnp.int32, (_S, _S), 1)
    tri = (jj <= ss).astype(jnp.float32)
    cum_row = jnp.dot(dur_row, tri, preferred_element_type=jnp.float32)
    t_col = lax.broadcasted_iota(jnp.int32, (_ML, 1), 0).astype(jnp.float32)

    src = jnp.sum((cum_row <= t_col).astype(jnp.float32), axis=-1, keepdims=True)
    mel_f = cum_row[:, _S - 1:]
    valid = t_col < jnp.minimum(mel_f, jnp.float32(_ML))
    base_f = (b * _S).astype(jnp.float32)
    ti = lax.broadcasted_iota(jnp.int32, (_ML, 1), 0)
    zrow = _ZROW + jnp.bitwise_and(ti, 63)
    idx_ref[0, 0] = jnp.where(valid, (src + base_f).astype(jnp.int32), zrow)

    pidx = jnp.sum((pbins_ref[:, :] < pt_ref[0]).astype(jnp.float32),
                   axis=-1, keepdims=True)
    eidx = jnp.sum((ebins_ref[:, :] < et_ref[0]).astype(jnp.float32),
                   axis=-1, keepdims=True)
    idx_ref[1, 0] = pidx.astype(jnp.int32)
    idx_ref[2, 0] = eidx.astype(jnp.int32)
    mel_ref[0] = cum_row[:, _S - 128:]


def _sc_xout_body(xpad_hbm, pemb_hbm, eemb_hbm, idx_hbm,
                  xout_hbm, gix, pix, eix, *bufs_and_sems):
    xbufs = bufs_and_sems[0:_NBUF]
    pbufs = bufs_and_sems[_NBUF:2 * _NBUF]
    ebufs = bufs_and_sems[2 * _NBUF:3 * _NBUF]
    gsems = bufs_and_sems[3 * _NBUF:4 * _NBUF]
    wsems = bufs_and_sems[4 * _NBUF:5 * _NBUF]
    wid = lax.axis_index("s") * 2 + lax.axis_index("c")
    pltpu.sync_copy(idx_hbm.at[0, wid], gix)
    pltpu.sync_copy(idx_hbm.at[1, wid], pix)
    pltpu.sync_copy(idx_hbm.at[2, wid], eix)

    def fire(k):
        m = k % _NBUF
        pltpu.async_copy(xpad_hbm.at[gix.at[k]], xbufs[m], gsems[m])
        pltpu.async_copy(pemb_hbm.at[pix.at[k]], pbufs[m], gsems[m])
        pltpu.async_copy(eemb_hbm.at[eix.at[k]], ebufs[m], gsems[m])

    def wait_gather(k):
        m = k % _NBUF
        pltpu.make_async_copy(xpad_hbm.at[gix.at[k]], xbufs[m], gsems[m]).wait()
        pltpu.make_async_copy(pemb_hbm.at[pix.at[k]], pbufs[m], gsems[m]).wait()
        pltpu.make_async_copy(eemb_hbm.at[eix.at[k]], ebufs[m], gsems[m]).wait()

    def out_at(k):
        return xout_hbm.at[pl.ds(wid * _RPW + k * _CH, _CH)]

    def wait_write(k):
        pltpu.make_async_copy(xbufs[k % _NBUF], out_at(k), wsems[k % _NBUF]).wait()

    for k in range(_NBUF - 1):
        fire(k)
    for k in range(_NCH):
        m = k % _NBUF
        if k + _NBUF - 1 < _NCH:
            if k >= 1:
                wait_write(k - 1)
            fire(k + _NBUF - 1)
        wait_gather(k)

        def row(r, c):
            for j in range(_D // 16):
                sl = pl.ds(j * 16, 16)
                xbufs[m][r, sl] = xbufs[m][r, sl] + pbufs[m][r, sl] + ebufs[m][r, sl]
            return c

        lax.fori_loop(0, _CH, row, 0)
        pltpu.async_copy(xbufs[m], out_at(k), wsems[m])
    for k in range(_NCH - _NBUF, _NCH):
        if k >= 0:
            wait_write(k)


def _main_body(xf_ref, dur_ref,
               dw1, db1, dg1, dbe1, dw2, db2, dg2, dbe2, dlwt, dlb,
               pw1, pb1, pg1, pbe1, pw2, pb2, pg2, pbe2, plwt, plb,
               ew1, eb1, eg1, ebe1, ew2, eb2, eg2, ebe2, elwt, elb,
               logdur_ref, ppred_ref, epred_ref):
    xf_b = xf_ref[0]
    dur_row = dur_ref[0].astype(jnp.float32)

    logdur_ref[0] = _predictor(xf_b, dw1, db1, dg1, dbe1, dw2, db2, dg2, dbe2, dlwt, dlb)

    _, gmat = _cum_and_gmat(dur_row)
    xf_exp = jnp.dot(gmat, xf_b, preferred_element_type=jnp.float32)

    ppred_ref[0] = _predictor(xf_exp, pw1, pb1, pg1, pbe1, pw2, pb2, pg2, pbe2, plwt, plb)
    epred_ref[0] = _predictor(xf_exp, ew1, eb1, eg1, ebe1, ew2, eb2, eg2, ebe2, elwt, elb)


def kernel(x, x_features, src_mask, mel_mask, duration_target, pitch_target,
           energy_target, max_len, dur_params, pitch_params, energy_params,
           pitch_bins, energy_bins, pitch_embedding, energy_embedding):
    B, S, D = x.shape
    ML = mel_mask.shape[1]

    dur3 = duration_target.reshape(B, 1, S)
    pt3 = pitch_target.reshape(B, ML, 1)
    et3 = energy_target.reshape(B, ML, 1)
    pad = jnp.full((1,), jnp.inf, jnp.float32)
    pbins = jnp.concatenate([pitch_bins, pad]).reshape(1, 256)
    ebins = jnp.concatenate([energy_bins, pad]).reshape(1, 256)

    batch3 = lambda i: (i, 0, 0)
    dparams = _pack_params(dur_params)
    pparams = _pack_params(pitch_params)
    eparams = _pack_params(energy_params)

    mel3, idx3 = pl.pallas_call(
        _idx_body,
        grid=(B,),
        in_specs=[
            pl.BlockSpec((1, 1, S), batch3),
            pl.BlockSpec((1, ML, 1), batch3),
            pl.BlockSpec((1, ML, 1), batch3),
            _full_spec(pbins), _full_spec(ebins),
        ],
        out_specs=[
            pl.BlockSpec((1, 1, 128), batch3),
            pl.BlockSpec((3, 1, ML, 1), lambda i: (0, i, 0, 0)),
        ],
        out_shape=[
            jax.ShapeDtypeStruct((B, 1, 128), jnp.float32),
            jax.ShapeDtypeStruct((3, B, ML, 1), jnp.int32),
        ],
    )(dur3, pt3, et3, pbins, ebins)

    xpad = jnp.pad(x.reshape(B * S, D), ((0, 64), (0, 0)))
    idx_w = idx3.reshape(3, _NW, _NCH, _CH)

    mesh = plsc.VectorSubcoreMesh(core_axis_name="c", subcore_axis_name="s")

    xout_flat = functools.partial(
        pl.kernel,
        out_type=jax.ShapeDtypeStruct((_NROWS, D), jnp.float32),
        mesh=mesh,
        scratch_types=(
            [pltpu.VMEM((_NCH, _CH), jnp.int32)] * 3
            + [pltpu.VMEM((_CH, D), jnp.float32)] * (3 * _NBUF)
            + [pltpu.SemaphoreType.DMA] * (2 * _NBUF)
        ),
    )(_sc_xout_body)(xpad, pitch_embedding, energy_embedding, idx_w)

    logdur3, ppred3, epred3 = pl.pallas_call(
        _main_body,
        grid=(B,),
        in_specs=[
            pl.BlockSpec((1, S, D), batch3),
            pl.BlockSpec((1, 1, S), batch3),
        ] + [_full_spec(a) for a in dparams + pparams + eparams],
        out_specs=[
            pl.BlockSpec((1, S, 1), batch3),
            pl.BlockSpec((1, ML, 1), batch3),
            pl.BlockSpec((1, ML, 1), batch3),
        ],
        out_shape=[
            jax.ShapeDtypeStruct((B, S, 1), jnp.float32),
            jax.ShapeDtypeStruct((B, ML, 1), jnp.float32),
            jax.ShapeDtypeStruct((B, ML, 1), jnp.float32),
        ],
    )(x_features, dur3, *dparams, *pparams, *eparams)

    x_out = xout_flat.reshape(B, ML, D)
    log_duration_prediction = logdur3.reshape(B, S)
    pitch_prediction = ppred3.reshape(B, ML)
    energy_prediction = epred3.reshape(B, ML)
    mel_len = mel3[:, 0, 127].astype(jnp.int32)

    return (x_out, log_duration_prediction, duration_target, pitch_prediction,
            energy_prediction, mel_len, mel_mask)
